# baseline (speedup 1.0000x reference)
import functools

import jax
import jax.numpy as jnp
from jax import lax
from jax.experimental import pallas as pl
from jax.experimental.pallas import tpu as pltpu
from jax.experimental.pallas import tpu_sc as plsc

N = 10000
NUM_EDGES = 640000
Z = 1000
ZE = 16000
D = 32
H = 2
T = 5
FF = 128

XW = 2 * D
PW = XW + 16
RB = 400
NBLK = N // RB

SC_NC = 2
SC_NS = 16
NW = SC_NC * SC_NS
CHUNK = 128
FULL_CHUNKS = NUM_EDGES // (NW * CHUNK)
EPW = FULL_CHUNKS * CHUNK
TAIL_BASE = NW * EPW
TAIL_CHUNKS = (NUM_EDGES - TAIL_BASE) // CHUNK
NPAD = 10240
ROWS_PER_TILE = NPAD // SC_NS


def _zone_const_body(ls_ref, ld_ref, e_ref, wg_ref, bg_ref,
                     allowed_ref, zctx_ref, en_ref, c_ref):
    i = pl.program_id(0)
    nsteps = pl.num_programs(0)

    @pl.when(i == 0)
    def _():
        c_ref[...] = jnp.zeros_like(c_ref)

    ls = ls_ref[...].reshape(1, -1)
    ld = ld_ref[...]
    iz_row = lax.broadcasted_iota(jnp.int32, (Z, ls.shape[1]), 0)
    os_t = (iz_row == ls).astype(jnp.bfloat16)
    iz_lane = lax.broadcasted_iota(jnp.int32, (ld.shape[0], Z), 1)
    od = (iz_lane == ld).astype(jnp.bfloat16)
    c_ref[...] += jnp.dot(os_t, od, preferred_element_type=jnp.float32)

    @pl.when(i == nsteps - 1)
    def _():
        C = c_ref[...]
        r0 = lax.broadcasted_iota(jnp.int32, (Z, Z), 0)
        r1 = lax.broadcasted_iota(jnp.int32, (Z, Z), 1)
        diag = r0 == r1
        allowed_ref[...] = ((C > 0.0) | diag).astype(jnp.bfloat16)
        E = e_ref[...]
        ones_col = jnp.ones((Z, 1), jnp.float32)
        deg = lax.dot_general(C, ones_col, (((0,), (0,)), ((), ()))) + 1.0
        dinv = 1.0 / jnp.sqrt(deg)
        ews = jnp.dot(E, wg_ref[...], preferred_element_type=jnp.float32)
        S = ews * dinv
        t1 = lax.dot_general(C, S, (((0,), (0,)), ((), ()))) + S
        zctx_ref[...] = dinv * t1 + bg_ref[...]
        e2 = E * E
        en_ref[...] = lax.dot_general(
            jnp.ones((1, D), jnp.float32), e2, (((1,), (1,)), ((), ())))


def _zone_consts(lsrc, ldst, E, W_gcn, b_gcn):
    EB = 1000
    grid = ZE // EB
    return pl.pallas_call(
        _zone_const_body,
        grid=(grid,),
        in_specs=[
            pl.BlockSpec((1, 1, EB), lambda i: (i, 0, 0)),
            pl.BlockSpec((EB, 1), lambda i: (i, 0)),
            pl.BlockSpec((Z, D), lambda i: (0, 0)),
            pl.BlockSpec((D, D), lambda i: (0, 0)),
            pl.BlockSpec((1, D), lambda i: (0, 0)),
        ],
        out_specs=[
            pl.BlockSpec((Z, Z), lambda i: (0, 0)),
            pl.BlockSpec((Z, D), lambda i: (0, 0)),
            pl.BlockSpec((1, Z), lambda i: (0, 0)),
        ],
        out_shape=[
            jax.ShapeDtypeStruct((Z, Z), jnp.bfloat16),
            jax.ShapeDtypeStruct((Z, D), jnp.float32),
            jax.ShapeDtypeStruct((1, Z), jnp.float32),
        ],
        scratch_shapes=[pltpu.VMEM((Z, Z), jnp.float32)],
    )(lsrc.reshape(ZE // EB, 1, EB), ldst.reshape(ZE, 1), E, W_gcn,
      b_gcn.reshape(1, D))


def _node_const_body(x_ref, e_ref, wpe_ref, bpe_ref, wgat_ref, axs_ref, aad_ref,
                     init_ref, base_ref, xwa_ref, ad_ref):
    xv = x_ref[...]
    xi = xv.astype(jnp.int32)
    iz = lax.broadcasted_iota(jnp.int32, (RB, Z), 1)
    P = (iz == xi).astype(jnp.float32)
    init = jnp.dot(P, e_ref[...], preferred_element_type=jnp.float32)
    init_ref[...] = init
    base_ref[...] = xv * wpe_ref[...] + bpe_ref[...]
    xw = jnp.dot(init, wgat_ref[...], preferred_element_type=jnp.float32)
    extra = jnp.dot(xw, axs_ref[...], preferred_element_type=jnp.float32)
    xwa_ref[...] = jnp.concatenate([xw, extra], axis=1)
    ad_ref[...] = jnp.dot(xw, aad_ref[...], preferred_element_type=jnp.float32)


def _node_consts(x, E, W_pe, b_pe, W_gat, Axs, Aad):
    return pl.pallas_call(
        _node_const_body,
        grid=(NBLK,),
        in_specs=[
            pl.BlockSpec((RB, 1), lambda i: (i, 0)),
            pl.BlockSpec((Z, D), lambda i: (0, 0)),
            pl.BlockSpec((1, D), lambda i: (0, 0)),
            pl.BlockSpec((1, D), lambda i: (0, 0)),
            pl.BlockSpec((D, XW), lambda i: (0, 0)),
            pl.BlockSpec((XW, 16), lambda i: (0, 0)),
            pl.BlockSpec((XW, 16), lambda i: (0, 0)),
        ],
        out_specs=[
            pl.BlockSpec((RB, D), lambda i: (i, 0)),
            pl.BlockSpec((RB, D), lambda i: (i, 0)),
            pl.BlockSpec((RB, PW), lambda i: (i, 0)),
            pl.BlockSpec((RB, 16), lambda i: (i, 0)),
        ],
        out_shape=[
            jax.ShapeDtypeStruct((N, D), jnp.float32),
            jax.ShapeDtypeStruct((N, D), jnp.float32),
            jax.ShapeDtypeStruct((N, PW), jnp.float32),
            jax.ShapeDtypeStruct((N, 16), jnp.float32),
        ],
    )(x, E, W_pe.reshape(1, D), b_pe.reshape(1, D), W_gat, Axs, Aad)


def _gat_edge_body(src_hbm, dst_hbm, xwa_hbm, ad_hbm, zeros_hbm, out_hbm,
                   srcv0a, srcv0b, srcv0c, dstv0a, dstv0b, dstv0c,
                   xwav0, adv0, outv0,
                   srcv1a, srcv1b, srcv1c, dstv1a, dstv1b, dstv1c,
                   xwav1, adv1, outv1,
                   comb, gsem0, gsem1, ssem0, ssem1, isem0, isem1):
    c = lax.axis_index("c")
    s = lax.axis_index("s")
    wid = s * SC_NC + c

    pltpu.sync_copy(zeros_hbm, comb.at[pl.ds(s * ROWS_PER_TILE, ROWS_PER_TILE)])
    plsc.subcore_barrier()

    iota = lax.iota(jnp.int32, 16)
    lane2 = iota < 2
    bdn = lax.GatherDimensionNumbers(
        offset_dims=(), collapsed_slice_dims=(0,), start_index_map=(0,))

    def lane_bcast(v, k):
        return lax.gather(v, (iota * 0 + k)[:, None], bdn, (1,),
                          mode=lax.GatherScatterMode.PROMISE_IN_BOUNDS)

    slots = (((srcv0a, srcv0b, srcv0c), (dstv0a, dstv0b, dstv0c),
              xwav0, adv0, outv0, gsem0, ssem0, isem0),
             ((srcv1a, srcv1b, srcv1c), (dstv1a, dstv1b, dstv1c),
              xwav1, adv1, outv1, gsem1, ssem1, isem1))

    def fetch_idx(base, srcv, dstv, isem):
        pltpu.async_copy(src_hbm.at[pl.ds(base, CHUNK)], srcv, isem)
        pltpu.async_copy(dst_hbm.at[pl.ds(base, CHUNK)], dstv, isem)

    def wait_idx(srcv, dstv, isem):
        pltpu.make_async_copy(src_hbm.at[pl.ds(0, CHUNK)], srcv, isem).wait()
        pltpu.make_async_copy(dst_hbm.at[pl.ds(0, CHUNK)], dstv, isem).wait()

    def start_gathers(srcv, dstv, xwav, adv, gsem):
        pltpu.async_copy(xwa_hbm.at[srcv], xwav, gsem)
        pltpu.async_copy(ad_hbm.at[dstv], adv, gsem)

    def wait_gathers(xwav, adv, gsem):
        pltpu.make_async_copy(xwa_hbm.at[srcv0a], xwav, gsem).wait()
        pltpu.make_async_copy(ad_hbm.at[dstv0a], adv, gsem).wait()

    def wait_scatter(outv, ssem):
        pltpu.make_async_copy(xwa_hbm.at[srcv0a], outv, ssem).wait()

    def compute(xwav, adv, outv):
        def edge_body(e8, carry):
            for k in range(8):
                e = e8 * 8 + k
                xa = xwav[e, pl.ds(XW, 16)]
                ad16 = adv[e, pl.ds(0, 16)]
                sv = jnp.where(lane2, xa + ad16, -1e4)
                ex = jnp.exp(jnp.where(sv >= 0.0, sv, 0.2 * sv))
                e0 = lane_bcast(ex, 0)
                e1 = lane_bcast(ex, 1)
                outv[e, pl.ds(0, 16)] = xwav[e, pl.ds(0, 16)] * e0
                outv[e, pl.ds(16, 16)] = xwav[e, pl.ds(16, 16)] * e0
                outv[e, pl.ds(32, 16)] = xwav[e, pl.ds(32, 16)] * e1
                outv[e, pl.ds(48, 16)] = xwav[e, pl.ds(48, 16)] * e1
                outv[e, pl.ds(XW, 16)] = ex
            return carry

        lax.fori_loop(0, CHUNK // 8, edge_body, 0)

    def do_pair(pj, par):
        cur, nxt, nn2 = par, (par + 1) % 3, (par + 2) % 3
        for b in range(2):
            srcs, dsts, xwav, adv, outv, gsem, ssem, isem = slots[b]
            wait_gathers(xwav, adv, gsem)
            wait_scatter(outv, ssem)
            compute(xwav, adv, outv)
            pltpu.async_copy(outv, comb.at[dsts[cur]], ssem, add=True)
            wait_idx(srcs[nxt], dsts[nxt], isem)
            start_gathers(srcs[nxt], dsts[nxt], xwav, adv, gsem)
            nbase = wid * EPW + (pj + 2) * (2 * CHUNK) + b * CHUNK
            nbase = jnp.minimum(nbase, NUM_EDGES - CHUNK)
            fetch_idx(nbase, srcs[nn2], dsts[nn2], isem)

    def pair3_body(q, carry):
        do_pair(3 * q, 0)
        do_pair(3 * q + 1, 1)
        do_pair(3 * q + 2, 2)
        return carry

    for b in range(2):
        srcs, dsts, xwav, adv, outv, gsem, ssem, isem = slots[b]
        pltpu.sync_copy(src_hbm.at[pl.ds(wid * EPW + b * CHUNK, CHUNK)],
                        srcs[0])
        pltpu.sync_copy(dst_hbm.at[pl.ds(wid * EPW + b * CHUNK, CHUNK)],
                        dsts[0])
        start_gathers(srcs[0], dsts[0], xwav, adv, gsem)
        fetch_idx(wid * EPW + 2 * CHUNK + b * CHUNK, srcs[1], dsts[1], isem)
        pltpu.async_copy(zeros_hbm.at[pl.ds(0, CHUNK)], outv, ssem)

    lax.fori_loop(0, FULL_CHUNKS // 6, pair3_body, 0)

    for b in range(2):
        srcs, dsts, xwav, adv, outv, gsem, ssem, isem = slots[b]
        wait_gathers(xwav, adv, gsem)
        wait_scatter(outv, ssem)
        wait_idx(srcs[0], dsts[0], isem)

    @pl.when(wid < TAIL_CHUNKS)
    def _():
        srcs, dsts, xwav, adv, outv, gsem, ssem, isem = slots[0]
        base = TAIL_BASE + wid * CHUNK
        pltpu.sync_copy(src_hbm.at[pl.ds(base, CHUNK)], srcs[0])
        pltpu.sync_copy(dst_hbm.at[pl.ds(base, CHUNK)], dsts[0])
        start_gathers(srcs[0], dsts[0], xwav, adv, gsem)
        wait_gathers(xwav, adv, gsem)
        compute(xwav, adv, outv)
        pltpu.sync_copy(outv, comb.at[dsts[0]], add=True)

    plsc.subcore_barrier()
    pltpu.sync_copy(comb.at[pl.ds(s * ROWS_PER_TILE, ROWS_PER_TILE)],
                    out_hbm.at[c, pl.ds(s * ROWS_PER_TILE, ROWS_PER_TILE)])


@functools.lru_cache(maxsize=1)
def _gat_edge_pass_fn():
    return functools.partial(
        pl.kernel,
        mesh=plsc.VectorSubcoreMesh(core_axis_name="c", subcore_axis_name="s"),
        compiler_params=pltpu.CompilerParams(use_tc_tiling_on_sc=False,
                                             needs_layout_passes=False),
        out_type=jax.ShapeDtypeStruct((SC_NC, NPAD, PW), jnp.float32),
        scratch_types=[
            pltpu.VMEM((CHUNK,), jnp.int32),
            pltpu.VMEM((CHUNK,), jnp.int32),
            pltpu.VMEM((CHUNK,), jnp.int32),
            pltpu.VMEM((CHUNK,), jnp.int32),
            pltpu.VMEM((CHUNK,), jnp.int32),
            pltpu.VMEM((CHUNK,), jnp.int32),
            pltpu.VMEM((CHUNK, PW), jnp.float32),
            pltpu.VMEM((CHUNK, 16), jnp.float32),
            pltpu.VMEM((CHUNK, PW), jnp.float32),
            pltpu.VMEM((CHUNK,), jnp.int32),
            pltpu.VMEM((CHUNK,), jnp.int32),
            pltpu.VMEM((CHUNK,), jnp.int32),
            pltpu.VMEM((CHUNK,), jnp.int32),
            pltpu.VMEM((CHUNK,), jnp.int32),
            pltpu.VMEM((CHUNK,), jnp.int32),
            pltpu.VMEM((CHUNK, PW), jnp.float32),
            pltpu.VMEM((CHUNK, 16), jnp.float32),
            pltpu.VMEM((CHUNK, PW), jnp.float32),
            pltpu.VMEM_SHARED((NPAD, PW), jnp.float32),
            pltpu.SemaphoreType.DMA,
            pltpu.SemaphoreType.DMA,
            pltpu.SemaphoreType.DMA,
            pltpu.SemaphoreType.DMA,
            pltpu.SemaphoreType.DMA,
            pltpu.SemaphoreType.DMA,
        ],
    )(_gat_edge_body)


def _gat_edge_pass(src, dst, xwa, ad, zeros_sc):
    return _gat_edge_pass_fn()(src, dst, xwa, ad, zeros_sc)


def _main_body(y_ref, xwa_ref, comb_ref, base_ref, allowed_ref, e_ref, en_ref,
               zctx_ref, w1_ref, b1_ref, w2_ref, b2_ref, wt_ref, bt_ref,
               bgat_ref, wgat_ref, axs_ref, aad_ref, tv_ref,
               ynext_ref, xwan_ref, adn_ref):
    y = y_ref[...]
    xwa = xwa_ref[...]
    cmb = comb_ref[...]
    acc = cmb[0] + cmb[1]

    asn = xwa[:, XW:XW + 2]
    adn = xwa[:, XW + 2:XW + 4]
    ssl = asn + adn
    ex_self = jnp.exp(jnp.where(ssl >= 0.0, ssl, 0.2 * ssl))

    ssum0 = acc[:, XW:XW + 1] + ex_self[:, 0:1] + 1e-16
    ssum1 = acc[:, XW + 1:XW + 2] + ex_self[:, 1:2] + 1e-16
    num0 = acc[:, 0:D] + ex_self[:, 0:1] * xwa[:, 0:D]
    num1 = acc[:, D:XW] + ex_self[:, 1:2] * xwa[:, D:XW]
    social = 0.5 * (num0 / ssum0 + num1 / ssum1) + bgat_ref[...]

    E = e_ref[...]
    En = en_ref[...]
    yE = lax.dot_general(y, E, (((1,), (1,)), ((), ())))
    d2m = jnp.sum(y * y, axis=1, keepdims=True) - 2.0 * yE + En
    m = jnp.min(d2m, axis=1, keepdims=True)
    il = lax.broadcasted_iota(jnp.int32, (RB, Z), 1)
    zi = jnp.min(jnp.where(d2m == m, il, Z), axis=1, keepdims=True)
    Pf = (il == zi).astype(jnp.float32)
    Pb = (il == zi).astype(jnp.bfloat16)

    Ezi = jnp.dot(Pf, E, preferred_element_type=jnp.float32)
    En_zi = jnp.sum(Pf * En, axis=1, keepdims=True)
    loc = jnp.dot(Pf, zctx_ref[...], preferred_element_type=jnp.float32)
    maskf = jnp.dot(Pb, allowed_ref[...], preferred_element_type=jnp.float32)

    t = tv_ref[0, 0]
    dt = tv_ref[0, 1]
    t_enc = t * wt_ref[...] + bt_ref[...]
    fi = jnp.concatenate(
        [social, loc, base_ref[...], jnp.broadcast_to(t_enc, (RB, D))], axis=1)
    h1 = jnp.maximum(
        jnp.dot(fi, w1_ref[...], preferred_element_type=jnp.float32)
        + b1_ref[...], 0.0)
    desired = jnp.dot(h1, w2_ref[...], preferred_element_type=jnp.float32) \
        + b2_ref[...]

    A = lax.dot_general(desired, E, (((1,), (1,)), ((), ())))
    bsel = jnp.sum(desired * Ezi, axis=1, keepdims=True)
    dz2 = En_zi + En - 2.0 * lax.dot_general(Ezi, E, (((1,), (1,)), ((), ())))
    okz = dz2 > 1e-12
    Dn = jnp.where(okz, jnp.sqrt(jnp.where(okz, dz2, 1.0)), 0.0)
    safe = jnp.where(Dn > 1e-6, Dn, 1.0)
    proj = (A - bsel) / safe
    is_allowed = maskf > 0.5
    lmax = jnp.max(jnp.where(is_allowed, proj, -jnp.inf), axis=1, keepdims=True)
    ez = jnp.where(is_allowed, jnp.exp(proj - lmax), 0.0)
    w = ez / jnp.sum(ez, axis=1, keepdims=True)
    u = w / safe
    su = jnp.sum(u, axis=1, keepdims=True)
    fv = jnp.dot(u, E, preferred_element_type=jnp.float32) - su * Ezi
    ynew = y + dt * (0.1 * fv)
    ynext_ref[...] = ynew

    xw = jnp.dot(ynew, wgat_ref[...], preferred_element_type=jnp.float32)
    extra = jnp.dot(xw, axs_ref[...], preferred_element_type=jnp.float32)
    xwan_ref[...] = jnp.concatenate([xw, extra], axis=1)
    adn_ref[...] = jnp.dot(xw, aad_ref[...], preferred_element_type=jnp.float32)


def _main_step(y, xwa, comb, base, allowed_bf, E, En, zctx,
               W1, b1, W2, b2, W_t, b_t, b_gat, W_gat, Axs, Aad, tv):
    return pl.pallas_call(
        _main_body,
        grid=(NBLK,),
        in_specs=[
            pl.BlockSpec((RB, D), lambda i: (i, 0)),
            pl.BlockSpec((RB, PW), lambda i: (i, 0)),
            pl.BlockSpec((2, RB, PW), lambda i: (0, i, 0)),
            pl.BlockSpec((RB, D), lambda i: (i, 0)),
            pl.BlockSpec((Z, Z), lambda i: (0, 0)),
            pl.BlockSpec((Z, D), lambda i: (0, 0)),
            pl.BlockSpec((1, Z), lambda i: (0, 0)),
            pl.BlockSpec((Z, D), lambda i: (0, 0)),
            pl.BlockSpec((FF, FF), lambda i: (0, 0)),
            pl.BlockSpec((1, FF), lambda i: (0, 0)),
            pl.BlockSpec((FF, D), lambda i: (0, 0)),
            pl.BlockSpec((1, D), lambda i: (0, 0)),
            pl.BlockSpec((1, D), lambda i: (0, 0)),
            pl.BlockSpec((1, D), lambda i: (0, 0)),
            pl.BlockSpec((1, D), lambda i: (0, 0)),
            pl.BlockSpec((D, XW), lambda i: (0, 0)),
            pl.BlockSpec((XW, 16), lambda i: (0, 0)),
            pl.BlockSpec((XW, 16), lambda i: (0, 0)),
            pl.BlockSpec(memory_space=pltpu.SMEM),
        ],
        out_specs=[
            pl.BlockSpec((RB, D), lambda i: (i, 0)),
            pl.BlockSpec((RB, PW), lambda i: (i, 0)),
            pl.BlockSpec((RB, 16), lambda i: (i, 0)),
        ],
        out_shape=[
            jax.ShapeDtypeStruct((N, D), jnp.float32),
            jax.ShapeDtypeStruct((N, PW), jnp.float32),
            jax.ShapeDtypeStruct((N, 16), jnp.float32),
        ],
    )(y, xwa, comb, base, allowed_bf, E, En, zctx,
      W1, b1.reshape(1, FF), W2, b2.reshape(1, D), W_t.reshape(1, D),
      b_t.reshape(1, D), b_gat.reshape(1, D), W_gat, Axs, Aad, tv)


def kernel(x, edge_index, loc_edge_index, times, E, W_pe, b_pe, W_gat, a_src,
           a_dst, b_gat, W_gcn, b_gcn, W_t, b_t, W1, b1, W2, b2):
    src = edge_index[0]
    dst = edge_index[1]
    lsrc = loc_edge_index[0]
    ldst = loc_edge_index[1]

    zero16 = jnp.zeros((D, 16), jnp.float32)
    Axs = jnp.concatenate([
        jnp.stack([a_src[0], jnp.zeros(D), a_dst[0], jnp.zeros(D)], axis=1),
        jnp.stack([jnp.zeros(D), a_src[1], jnp.zeros(D), a_dst[1]], axis=1),
    ], axis=0)
    Axs = jnp.concatenate([Axs, jnp.zeros((XW, 12), jnp.float32)], axis=1)
    Aad = jnp.concatenate([
        jnp.stack([a_dst[0], jnp.zeros(D)], axis=1),
        jnp.stack([jnp.zeros(D), a_dst[1]], axis=1),
    ], axis=0)
    Aad = jnp.concatenate([Aad, jnp.zeros((XW, 14), jnp.float32)], axis=1)

    allowed_bf, zctx, En = _zone_consts(lsrc, ldst, E, W_gcn, b_gcn)
    init, base, xwa, ad = _node_consts(x, E, W_pe, b_pe, W_gat, Axs, Aad)

    zeros_sc = jnp.zeros((ROWS_PER_TILE, PW), jnp.float32)

    traj = [init]
    y = init
    for k in range(T - 1):
        comb = _gat_edge_pass(src, dst, xwa, ad, zeros_sc)
        tv = jnp.stack([times[k], times[k + 1] - times[k]]).reshape(1, 2)
        y, xwa, ad = _main_step(y, xwa, comb, base, allowed_bf, E, En, zctx,
                                W1, b1, W2, b2, W_t, b_t, b_gat, W_gat,
                                Axs, Aad, tv)
        traj.append(y)
    return jnp.stack(traj, axis=0)


# speedup vs baseline: 1.0108x; 1.0108x over previous
import functools

import jax
import jax.numpy as jnp
from jax import lax
from jax.experimental import pallas as pl
from jax.experimental.pallas import tpu as pltpu
from jax.experimental.pallas import tpu_sc as plsc

N = 10000
NUM_EDGES = 640000
Z = 1000
ZE = 16000
D = 32
H = 2
T = 5
FF = 128

XW = 2 * D
PW = XW + 16
RB = 400
NBLK = N // RB

SC_NC = 2
SC_NS = 16
NW = SC_NC * SC_NS
CHUNK = 128
FULL_CHUNKS = NUM_EDGES // (NW * CHUNK)
EPW = FULL_CHUNKS * CHUNK
TAIL_BASE = NW * EPW
TAIL_CHUNKS = (NUM_EDGES - TAIL_BASE) // CHUNK
NPAD = 10240
ROWS_PER_TILE = NPAD // SC_NS


def _zone_const_body(ls_ref, ld_ref, e_ref, wg_ref, bg_ref,
                     allowed_ref, zctx_ref, en_ref, c_ref):
    i = pl.program_id(0)
    nsteps = pl.num_programs(0)

    @pl.when(i == 0)
    def _():
        c_ref[...] = jnp.zeros_like(c_ref)

    ls = ls_ref[...].reshape(1, -1)
    ld = ld_ref[...]
    iz_row = lax.broadcasted_iota(jnp.int32, (Z, ls.shape[1]), 0)
    os_t = (iz_row == ls).astype(jnp.bfloat16)
    iz_lane = lax.broadcasted_iota(jnp.int32, (ld.shape[0], Z), 1)
    od = (iz_lane == ld).astype(jnp.bfloat16)
    c_ref[...] += jnp.dot(os_t, od, preferred_element_type=jnp.float32)

    @pl.when(i == nsteps - 1)
    def _():
        C = c_ref[...]
        r0 = lax.broadcasted_iota(jnp.int32, (Z, Z), 0)
        r1 = lax.broadcasted_iota(jnp.int32, (Z, Z), 1)
        diag = r0 == r1
        allowed_ref[...] = ((C > 0.0) | diag).astype(jnp.bfloat16)
        E = e_ref[...]
        ones_col = jnp.ones((Z, 1), jnp.float32)
        deg = lax.dot_general(C, ones_col, (((0,), (0,)), ((), ()))) + 1.0
        dinv = 1.0 / jnp.sqrt(deg)
        ews = jnp.dot(E, wg_ref[...], preferred_element_type=jnp.float32)
        S = ews * dinv
        t1 = lax.dot_general(C, S, (((0,), (0,)), ((), ()))) + S
        zctx_ref[...] = dinv * t1 + bg_ref[...]
        e2 = E * E
        en_ref[...] = lax.dot_general(
            jnp.ones((1, D), jnp.float32), e2, (((1,), (1,)), ((), ())))


def _zone_consts(lsrc, ldst, E, W_gcn, b_gcn):
    EB = 1000
    grid = ZE // EB
    return pl.pallas_call(
        _zone_const_body,
        grid=(grid,),
        in_specs=[
            pl.BlockSpec((1, 1, EB), lambda i: (i, 0, 0)),
            pl.BlockSpec((EB, 1), lambda i: (i, 0)),
            pl.BlockSpec((Z, D), lambda i: (0, 0)),
            pl.BlockSpec((D, D), lambda i: (0, 0)),
            pl.BlockSpec((1, D), lambda i: (0, 0)),
        ],
        out_specs=[
            pl.BlockSpec((Z, Z), lambda i: (0, 0)),
            pl.BlockSpec((Z, D), lambda i: (0, 0)),
            pl.BlockSpec((1, Z), lambda i: (0, 0)),
        ],
        out_shape=[
            jax.ShapeDtypeStruct((Z, Z), jnp.bfloat16),
            jax.ShapeDtypeStruct((Z, D), jnp.float32),
            jax.ShapeDtypeStruct((1, Z), jnp.float32),
        ],
        scratch_shapes=[pltpu.VMEM((Z, Z), jnp.float32)],
    )(lsrc.reshape(ZE // EB, 1, EB), ldst.reshape(ZE, 1), E, W_gcn,
      b_gcn.reshape(1, D))


def _node_const_body(x_ref, e_ref, wpe_ref, bpe_ref, wgat_ref, axs_ref, aad_ref,
                     init_ref, base_ref, xwa_ref, ad_ref):
    xv = x_ref[...]
    xi = xv.astype(jnp.int32)
    iz = lax.broadcasted_iota(jnp.int32, (RB, Z), 1)
    P = (iz == xi).astype(jnp.float32)
    init = jnp.dot(P, e_ref[...], preferred_element_type=jnp.float32)
    init_ref[...] = init
    base_ref[...] = xv * wpe_ref[...] + bpe_ref[...]
    xw = jnp.dot(init, wgat_ref[...], preferred_element_type=jnp.float32)
    extra = jnp.dot(xw, axs_ref[...], preferred_element_type=jnp.float32)
    xwa_ref[...] = jnp.concatenate([xw, extra], axis=1)
    ad_ref[...] = jnp.dot(xw, aad_ref[...], preferred_element_type=jnp.float32)


def _node_consts(x, E, W_pe, b_pe, W_gat, Axs, Aad):
    return pl.pallas_call(
        _node_const_body,
        grid=(NBLK,),
        in_specs=[
            pl.BlockSpec((RB, 1), lambda i: (i, 0)),
            pl.BlockSpec((Z, D), lambda i: (0, 0)),
            pl.BlockSpec((1, D), lambda i: (0, 0)),
            pl.BlockSpec((1, D), lambda i: (0, 0)),
            pl.BlockSpec((D, XW), lambda i: (0, 0)),
            pl.BlockSpec((XW, 16), lambda i: (0, 0)),
            pl.BlockSpec((XW, 16), lambda i: (0, 0)),
        ],
        out_specs=[
            pl.BlockSpec((RB, D), lambda i: (i, 0)),
            pl.BlockSpec((RB, D), lambda i: (i, 0)),
            pl.BlockSpec((RB, PW), lambda i: (i, 0)),
            pl.BlockSpec((RB, 16), lambda i: (i, 0)),
        ],
        out_shape=[
            jax.ShapeDtypeStruct((N, D), jnp.float32),
            jax.ShapeDtypeStruct((N, D), jnp.float32),
            jax.ShapeDtypeStruct((N, PW), jnp.float32),
            jax.ShapeDtypeStruct((N, 16), jnp.float32),
        ],
    )(x, E, W_pe.reshape(1, D), b_pe.reshape(1, D), W_gat, Axs, Aad)


def _gat_edge_body(src_hbm, dst_hbm, xwa_hbm, ad_hbm, zeros_hbm, out_hbm,
                   srcv0a, srcv0b, srcv0c, dstv0a, dstv0b, dstv0c,
                   xwav0a, xwav0b, adv0a, adv0b, outv0,
                   srcv1a, srcv1b, srcv1c, dstv1a, dstv1b, dstv1c,
                   xwav1a, xwav1b, adv1a, adv1b, outv1,
                   comb, gsem0, gsem1, ssem0, ssem1, isem0, isem1):
    c = lax.axis_index("c")
    s = lax.axis_index("s")
    wid = s * SC_NC + c

    pltpu.sync_copy(zeros_hbm, comb.at[pl.ds(s * ROWS_PER_TILE, ROWS_PER_TILE)])
    plsc.subcore_barrier()

    iota = lax.iota(jnp.int32, 16)
    lane2 = iota < 2
    bdn = lax.GatherDimensionNumbers(
        offset_dims=(), collapsed_slice_dims=(0,), start_index_map=(0,))

    def lane_bcast(v, k):
        return lax.gather(v, (iota * 0 + k)[:, None], bdn, (1,),
                          mode=lax.GatherScatterMode.PROMISE_IN_BOUNDS)

    slots = (((srcv0a, srcv0b, srcv0c), (dstv0a, dstv0b, dstv0c),
              (xwav0a, xwav0b), (adv0a, adv0b), outv0, gsem0, ssem0, isem0),
             ((srcv1a, srcv1b, srcv1c), (dstv1a, dstv1b, dstv1c),
              (xwav1a, xwav1b), (adv1a, adv1b), outv1, gsem1, ssem1, isem1))

    def fetch_idx(base, srcv, dstv, isem):
        pltpu.async_copy(src_hbm.at[pl.ds(base, CHUNK)], srcv, isem)
        pltpu.async_copy(dst_hbm.at[pl.ds(base, CHUNK)], dstv, isem)

    def wait_idx(srcv, dstv, isem):
        pltpu.make_async_copy(src_hbm.at[pl.ds(0, CHUNK)], srcv, isem).wait()
        pltpu.make_async_copy(dst_hbm.at[pl.ds(0, CHUNK)], dstv, isem).wait()

    def start_gathers(srcv, dstv, xwav, adv, gsem):
        pltpu.async_copy(xwa_hbm.at[srcv], xwav, gsem)
        pltpu.async_copy(ad_hbm.at[dstv], adv, gsem)

    def wait_gathers(xwav, adv, gsem):
        pltpu.make_async_copy(xwa_hbm.at[srcv0a], xwav, gsem).wait()
        pltpu.make_async_copy(ad_hbm.at[dstv0a], adv, gsem).wait()

    def wait_scatter(outv, ssem):
        pltpu.make_async_copy(xwa_hbm.at[srcv0a], outv, ssem).wait()

    def compute(xwav, adv, outv):
        def edge_body(e4, carry):
            for k in range(4):
                e = e4 * 4 + k
                xa = xwav[e, pl.ds(XW, 16)]
                ad16 = adv[e, pl.ds(0, 16)]
                sv = jnp.where(lane2, xa + ad16, 0.0)
                ex = jnp.exp(jnp.where(sv >= 0.0, sv, 0.2 * sv))
                exm = jnp.where(lane2, ex, 0.0)
                e0 = lane_bcast(ex, 0)
                e1 = lane_bcast(ex, 1)
                outv[e, pl.ds(0, 16)] = xwav[e, pl.ds(0, 16)] * e0
                outv[e, pl.ds(16, 16)] = xwav[e, pl.ds(16, 16)] * e0
                outv[e, pl.ds(32, 16)] = xwav[e, pl.ds(32, 16)] * e1
                outv[e, pl.ds(48, 16)] = xwav[e, pl.ds(48, 16)] * e1
                outv[e, pl.ds(XW, 16)] = exm
            return carry

        lax.fori_loop(0, CHUNK // 4, edge_body, 0)

    def do_pair(pj, par, xb):
        cur, nxt, nn2 = par, (par + 1) % 3, (par + 2) % 3
        for b in range(2):
            srcs, dsts, xwavs, advs, outv, gsem, ssem, isem = slots[b]
            xwav, adv = xwavs[xb], advs[xb]
            wait_gathers(xwav, adv, gsem)
            wait_idx(srcs[nxt], dsts[nxt], isem)
            start_gathers(srcs[nxt], dsts[nxt], xwavs[1 - xb], advs[1 - xb],
                          gsem)
            wait_scatter(outv, ssem)
            compute(xwav, adv, outv)
            pltpu.async_copy(outv, comb.at[dsts[cur]], ssem, add=True)
            nbase = wid * EPW + (pj + 2) * (2 * CHUNK) + b * CHUNK
            nbase = jnp.minimum(nbase, NUM_EDGES - CHUNK)
            fetch_idx(nbase, srcs[nn2], dsts[nn2], isem)

    def pair6_body(q, carry):
        for r in range(6):
            do_pair(6 * q + r, r % 3, r % 2)
        return carry

    for b in range(2):
        srcs, dsts, xwavs, advs, outv, gsem, ssem, isem = slots[b]
        pltpu.sync_copy(src_hbm.at[pl.ds(wid * EPW + b * CHUNK, CHUNK)],
                        srcs[0])
        pltpu.sync_copy(dst_hbm.at[pl.ds(wid * EPW + b * CHUNK, CHUNK)],
                        dsts[0])
        start_gathers(srcs[0], dsts[0], xwavs[0], advs[0], gsem)
        fetch_idx(wid * EPW + 2 * CHUNK + b * CHUNK, srcs[1], dsts[1], isem)
        pltpu.async_copy(zeros_hbm.at[pl.ds(0, CHUNK)], outv, ssem)

    lax.fori_loop(0, FULL_CHUNKS // 12, pair6_body, 0)

    for b in range(2):
        srcs, dsts, xwavs, advs, outv, gsem, ssem, isem = slots[b]
        wait_gathers(xwavs[0], advs[0], gsem)
        wait_scatter(outv, ssem)
        wait_idx(srcs[0], dsts[0], isem)

    @pl.when(wid < TAIL_CHUNKS)
    def _():
        srcs, dsts, xwavs, advs, outv, gsem, ssem, isem = slots[0]
        base = TAIL_BASE + wid * CHUNK
        pltpu.sync_copy(src_hbm.at[pl.ds(base, CHUNK)], srcs[0])
        pltpu.sync_copy(dst_hbm.at[pl.ds(base, CHUNK)], dsts[0])
        start_gathers(srcs[0], dsts[0], xwavs[0], advs[0], gsem)
        wait_gathers(xwavs[0], advs[0], gsem)
        compute(xwavs[0], advs[0], outv)
        pltpu.sync_copy(outv, comb.at[dsts[0]], add=True)

    plsc.subcore_barrier()
    pltpu.sync_copy(comb.at[pl.ds(s * ROWS_PER_TILE, ROWS_PER_TILE)],
                    out_hbm.at[c, pl.ds(s * ROWS_PER_TILE, ROWS_PER_TILE)])


@functools.lru_cache(maxsize=1)
def _gat_edge_pass_fn():
    return functools.partial(
        pl.kernel,
        mesh=plsc.VectorSubcoreMesh(core_axis_name="c", subcore_axis_name="s"),
        compiler_params=pltpu.CompilerParams(use_tc_tiling_on_sc=False,
                                             needs_layout_passes=False),
        out_type=jax.ShapeDtypeStruct((SC_NC, NPAD, PW), jnp.float32),
        scratch_types=[
            pltpu.VMEM((CHUNK,), jnp.int32),
            pltpu.VMEM((CHUNK,), jnp.int32),
            pltpu.VMEM((CHUNK,), jnp.int32),
            pltpu.VMEM((CHUNK,), jnp.int32),
            pltpu.VMEM((CHUNK,), jnp.int32),
            pltpu.VMEM((CHUNK,), jnp.int32),
            pltpu.VMEM((CHUNK, PW), jnp.float32),
            pltpu.VMEM((CHUNK, PW), jnp.float32),
            pltpu.VMEM((CHUNK, 16), jnp.float32),
            pltpu.VMEM((CHUNK, 16), jnp.float32),
            pltpu.VMEM((CHUNK, PW), jnp.float32),
            pltpu.VMEM((CHUNK,), jnp.int32),
            pltpu.VMEM((CHUNK,), jnp.int32),
            pltpu.VMEM((CHUNK,), jnp.int32),
            pltpu.VMEM((CHUNK,), jnp.int32),
            pltpu.VMEM((CHUNK,), jnp.int32),
            pltpu.VMEM((CHUNK,), jnp.int32),
            pltpu.VMEM((CHUNK, PW), jnp.float32),
            pltpu.VMEM((CHUNK, PW), jnp.float32),
            pltpu.VMEM((CHUNK, 16), jnp.float32),
            pltpu.VMEM((CHUNK, 16), jnp.float32),
            pltpu.VMEM((CHUNK, PW), jnp.float32),
            pltpu.VMEM_SHARED((NPAD, PW), jnp.float32),
            pltpu.SemaphoreType.DMA,
            pltpu.SemaphoreType.DMA,
            pltpu.SemaphoreType.DMA,
            pltpu.SemaphoreType.DMA,
            pltpu.SemaphoreType.DMA,
            pltpu.SemaphoreType.DMA,
        ],
    )(_gat_edge_body)


def _gat_edge_pass(src, dst, xwa, ad, zeros_sc):
    return _gat_edge_pass_fn()(src, dst, xwa, ad, zeros_sc)


def _main_body(y_ref, xwa_ref, comb_ref, base_ref, allowed_ref, e_ref, en_ref,
               zctx_ref, w1_ref, b1_ref, w2_ref, b2_ref, wt_ref, bt_ref,
               bgat_ref, wgat_ref, axs_ref, aad_ref, tv_ref,
               ynext_ref, xwan_ref, adn_ref):
    y = y_ref[...]
    xwa = xwa_ref[...]
    cmb = comb_ref[...]
    acc = cmb[0] + cmb[1]

    asn = xwa[:, XW:XW + 2]
    adn = xwa[:, XW + 2:XW + 4]
    ssl = asn + adn
    ex_self = jnp.exp(jnp.where(ssl >= 0.0, ssl, 0.2 * ssl))

    ssum0 = acc[:, XW:XW + 1] + ex_self[:, 0:1] + 1e-16
    ssum1 = acc[:, XW + 1:XW + 2] + ex_self[:, 1:2] + 1e-16
    num0 = acc[:, 0:D] + ex_self[:, 0:1] * xwa[:, 0:D]
    num1 = acc[:, D:XW] + ex_self[:, 1:2] * xwa[:, D:XW]
    social = 0.5 * (num0 / ssum0 + num1 / ssum1) + bgat_ref[...]

    E = e_ref[...]
    En = en_ref[...]
    yE = lax.dot_general(y, E, (((1,), (1,)), ((), ())))
    d2m = jnp.sum(y * y, axis=1, keepdims=True) - 2.0 * yE + En
    m = jnp.min(d2m, axis=1, keepdims=True)
    il = lax.broadcasted_iota(jnp.int32, (RB, Z), 1)
    zi = jnp.min(jnp.where(d2m == m, il, Z), axis=1, keepdims=True)
    Pf = (il == zi).astype(jnp.float32)
    Pb = (il == zi).astype(jnp.bfloat16)

    Ezi = jnp.dot(Pf, E, preferred_element_type=jnp.float32)
    En_zi = jnp.sum(Pf * En, axis=1, keepdims=True)
    loc = jnp.dot(Pf, zctx_ref[...], preferred_element_type=jnp.float32)
    maskf = jnp.dot(Pb, allowed_ref[...], preferred_element_type=jnp.float32)

    t = tv_ref[0, 0]
    dt = tv_ref[0, 1]
    t_enc = t * wt_ref[...] + bt_ref[...]
    fi = jnp.concatenate(
        [social, loc, base_ref[...], jnp.broadcast_to(t_enc, (RB, D))], axis=1)
    h1 = jnp.maximum(
        jnp.dot(fi, w1_ref[...], preferred_element_type=jnp.float32)
        + b1_ref[...], 0.0)
    desired = jnp.dot(h1, w2_ref[...], preferred_element_type=jnp.float32) \
        + b2_ref[...]

    A = lax.dot_general(desired, E, (((1,), (1,)), ((), ())))
    bsel = jnp.sum(desired * Ezi, axis=1, keepdims=True)
    dz2 = En_zi + En - 2.0 * lax.dot_general(Ezi, E, (((1,), (1,)), ((), ())))
    okz = dz2 > 1e-12
    Dn = jnp.where(okz, jnp.sqrt(jnp.where(okz, dz2, 1.0)), 0.0)
    safe = jnp.where(Dn > 1e-6, Dn, 1.0)
    proj = (A - bsel) / safe
    is_allowed = maskf > 0.5
    lmax = jnp.max(jnp.where(is_allowed, proj, -jnp.inf), axis=1, keepdims=True)
    ez = jnp.where(is_allowed, jnp.exp(proj - lmax), 0.0)
    w = ez / jnp.sum(ez, axis=1, keepdims=True)
    u = w / safe
    su = jnp.sum(u, axis=1, keepdims=True)
    fv = jnp.dot(u, E, preferred_element_type=jnp.float32) - su * Ezi
    ynew = y + dt * (0.1 * fv)
    ynext_ref[...] = ynew

    xw = jnp.dot(ynew, wgat_ref[...], preferred_element_type=jnp.float32)
    extra = jnp.dot(xw, axs_ref[...], preferred_element_type=jnp.float32)
    xwan_ref[...] = jnp.concatenate([xw, extra], axis=1)
    adn_ref[...] = jnp.dot(xw, aad_ref[...], preferred_element_type=jnp.float32)


def _main_step(y, xwa, comb, base, allowed_bf, E, En, zctx,
               W1, b1, W2, b2, W_t, b_t, b_gat, W_gat, Axs, Aad, tv):
    return pl.pallas_call(
        _main_body,
        grid=(NBLK,),
        in_specs=[
            pl.BlockSpec((RB, D), lambda i: (i, 0)),
            pl.BlockSpec((RB, PW), lambda i: (i, 0)),
            pl.BlockSpec((2, RB, PW), lambda i: (0, i, 0)),
            pl.BlockSpec((RB, D), lambda i: (i, 0)),
            pl.BlockSpec((Z, Z), lambda i: (0, 0)),
            pl.BlockSpec((Z, D), lambda i: (0, 0)),
            pl.BlockSpec((1, Z), lambda i: (0, 0)),
            pl.BlockSpec((Z, D), lambda i: (0, 0)),
            pl.BlockSpec((FF, FF), lambda i: (0, 0)),
            pl.BlockSpec((1, FF), lambda i: (0, 0)),
            pl.BlockSpec((FF, D), lambda i: (0, 0)),
            pl.BlockSpec((1, D), lambda i: (0, 0)),
            pl.BlockSpec((1, D), lambda i: (0, 0)),
            pl.BlockSpec((1, D), lambda i: (0, 0)),
            pl.BlockSpec((1, D), lambda i: (0, 0)),
            pl.BlockSpec((D, XW), lambda i: (0, 0)),
            pl.BlockSpec((XW, 16), lambda i: (0, 0)),
            pl.BlockSpec((XW, 16), lambda i: (0, 0)),
            pl.BlockSpec(memory_space=pltpu.SMEM),
        ],
        out_specs=[
            pl.BlockSpec((RB, D), lambda i: (i, 0)),
            pl.BlockSpec((RB, PW), lambda i: (i, 0)),
            pl.BlockSpec((RB, 16), lambda i: (i, 0)),
        ],
        out_shape=[
            jax.ShapeDtypeStruct((N, D), jnp.float32),
            jax.ShapeDtypeStruct((N, PW), jnp.float32),
            jax.ShapeDtypeStruct((N, 16), jnp.float32),
        ],
    )(y, xwa, comb, base, allowed_bf, E, En, zctx,
      W1, b1.reshape(1, FF), W2, b2.reshape(1, D), W_t.reshape(1, D),
      b_t.reshape(1, D), b_gat.reshape(1, D), W_gat, Axs, Aad, tv)


def kernel(x, edge_index, loc_edge_index, times, E, W_pe, b_pe, W_gat, a_src,
           a_dst, b_gat, W_gcn, b_gcn, W_t, b_t, W1, b1, W2, b2):
    src = edge_index[0]
    dst = edge_index[1]
    lsrc = loc_edge_index[0]
    ldst = loc_edge_index[1]

    zero16 = jnp.zeros((D, 16), jnp.float32)
    Axs = jnp.concatenate([
        jnp.stack([a_src[0], jnp.zeros(D), a_dst[0], jnp.zeros(D)], axis=1),
        jnp.stack([jnp.zeros(D), a_src[1], jnp.zeros(D), a_dst[1]], axis=1),
    ], axis=0)
    Axs = jnp.concatenate([Axs, jnp.zeros((XW, 12), jnp.float32)], axis=1)
    Aad = jnp.concatenate([
        jnp.stack([a_dst[0], jnp.zeros(D)], axis=1),
        jnp.stack([jnp.zeros(D), a_dst[1]], axis=1),
    ], axis=0)
    Aad = jnp.concatenate([Aad, jnp.zeros((XW, 14), jnp.float32)], axis=1)

    allowed_bf, zctx, En = _zone_consts(lsrc, ldst, E, W_gcn, b_gcn)
    init, base, xwa, ad = _node_consts(x, E, W_pe, b_pe, W_gat, Axs, Aad)

    zeros_sc = jnp.zeros((ROWS_PER_TILE, PW), jnp.float32)

    traj = [init]
    y = init
    for k in range(T - 1):
        comb = _gat_edge_pass(src, dst, xwa, ad, zeros_sc)
        tv = jnp.stack([times[k], times[k + 1] - times[k]]).reshape(1, 2)
        y, xwa, ad = _main_step(y, xwa, comb, base, allowed_bf, E, En, zctx,
                                W1, b1, W2, b2, W_t, b_t, b_gat, W_gat,
                                Axs, Aad, tv)
        traj.append(y)
    return jnp.stack(traj, axis=0)


# speedup vs baseline: 1.0169x; 1.0060x over previous
import functools

import jax
import jax.numpy as jnp
from jax import lax
from jax.experimental import pallas as pl
from jax.experimental.pallas import tpu as pltpu
from jax.experimental.pallas import tpu_sc as plsc

N = 10000
NUM_EDGES = 640000
Z = 1000
ZE = 16000
D = 32
H = 2
T = 5
FF = 128

XW = 2 * D
PW = XW + 16
RB = 400
NBLK = N // RB

SC_NC = 2
SC_NS = 16
NW = SC_NC * SC_NS
CHUNK = 128
FULL_CHUNKS = NUM_EDGES // (NW * CHUNK)
EPW = FULL_CHUNKS * CHUNK
TAIL_BASE = NW * EPW
TAIL_CHUNKS = (NUM_EDGES - TAIL_BASE) // CHUNK
NPAD = 10240
ROWS_PER_TILE = NPAD // SC_NS


def _zone_const_body(ls_ref, ld_ref, e_ref, wg_ref, bg_ref,
                     allowed_ref, zctx_ref, en_ref, c_ref):
    i = pl.program_id(0)
    nsteps = pl.num_programs(0)

    @pl.when(i == 0)
    def _():
        c_ref[...] = jnp.zeros_like(c_ref)

    ls = ls_ref[...].reshape(1, -1)
    ld = ld_ref[...]
    iz_row = lax.broadcasted_iota(jnp.int32, (Z, ls.shape[1]), 0)
    os_t = (iz_row == ls).astype(jnp.bfloat16)
    iz_lane = lax.broadcasted_iota(jnp.int32, (ld.shape[0], Z), 1)
    od = (iz_lane == ld).astype(jnp.bfloat16)
    c_ref[...] += jnp.dot(os_t, od, preferred_element_type=jnp.float32)

    @pl.when(i == nsteps - 1)
    def _():
        C = c_ref[...]
        r0 = lax.broadcasted_iota(jnp.int32, (Z, Z), 0)
        r1 = lax.broadcasted_iota(jnp.int32, (Z, Z), 1)
        diag = r0 == r1
        allowed_ref[...] = ((C > 0.0) | diag).astype(jnp.bfloat16)
        E = e_ref[...]
        ones_col = jnp.ones((Z, 1), jnp.float32)
        deg = lax.dot_general(C, ones_col, (((0,), (0,)), ((), ()))) + 1.0
        dinv = 1.0 / jnp.sqrt(deg)
        ews = jnp.dot(E, wg_ref[...], preferred_element_type=jnp.float32)
        S = ews * dinv
        t1 = lax.dot_general(C, S, (((0,), (0,)), ((), ()))) + S
        zctx_ref[...] = dinv * t1 + bg_ref[...]
        e2 = E * E
        en_ref[...] = lax.dot_general(
            jnp.ones((1, D), jnp.float32), e2, (((1,), (1,)), ((), ())))


def _zone_consts(lsrc, ldst, E, W_gcn, b_gcn):
    EB = 1000
    grid = ZE // EB
    return pl.pallas_call(
        _zone_const_body,
        grid=(grid,),
        in_specs=[
            pl.BlockSpec((1, 1, EB), lambda i: (i, 0, 0)),
            pl.BlockSpec((EB, 1), lambda i: (i, 0)),
            pl.BlockSpec((Z, D), lambda i: (0, 0)),
            pl.BlockSpec((D, D), lambda i: (0, 0)),
            pl.BlockSpec((1, D), lambda i: (0, 0)),
        ],
        out_specs=[
            pl.BlockSpec((Z, Z), lambda i: (0, 0)),
            pl.BlockSpec((Z, D), lambda i: (0, 0)),
            pl.BlockSpec((1, Z), lambda i: (0, 0)),
        ],
        out_shape=[
            jax.ShapeDtypeStruct((Z, Z), jnp.bfloat16),
            jax.ShapeDtypeStruct((Z, D), jnp.float32),
            jax.ShapeDtypeStruct((1, Z), jnp.float32),
        ],
        scratch_shapes=[pltpu.VMEM((Z, Z), jnp.float32)],
    )(lsrc.reshape(ZE // EB, 1, EB), ldst.reshape(ZE, 1), E, W_gcn,
      b_gcn.reshape(1, D))


def _node_const_body(x_ref, e_ref, wpe_ref, bpe_ref, wgat_ref, axs_ref, aad_ref,
                     init_ref, base_ref, xwa_ref, ad_ref):
    xv = x_ref[...]
    xi = xv.astype(jnp.int32)
    iz = lax.broadcasted_iota(jnp.int32, (RB, Z), 1)
    P = (iz == xi).astype(jnp.float32)
    init = jnp.dot(P, e_ref[...], preferred_element_type=jnp.float32)
    init_ref[...] = init
    base_ref[...] = xv * wpe_ref[...] + bpe_ref[...]
    xw = jnp.dot(init, wgat_ref[...], preferred_element_type=jnp.float32)
    extra = jnp.dot(xw, axs_ref[...], preferred_element_type=jnp.float32)
    xwa_ref[...] = jnp.concatenate([xw, extra], axis=1)
    ad_ref[...] = jnp.dot(xw, aad_ref[...], preferred_element_type=jnp.float32)


def _node_consts(x, E, W_pe, b_pe, W_gat, Axs, Aad):
    return pl.pallas_call(
        _node_const_body,
        grid=(NBLK,),
        in_specs=[
            pl.BlockSpec((RB, 1), lambda i: (i, 0)),
            pl.BlockSpec((Z, D), lambda i: (0, 0)),
            pl.BlockSpec((1, D), lambda i: (0, 0)),
            pl.BlockSpec((1, D), lambda i: (0, 0)),
            pl.BlockSpec((D, XW), lambda i: (0, 0)),
            pl.BlockSpec((XW, 16), lambda i: (0, 0)),
            pl.BlockSpec((XW, 16), lambda i: (0, 0)),
        ],
        out_specs=[
            pl.BlockSpec((RB, D), lambda i: (i, 0)),
            pl.BlockSpec((RB, D), lambda i: (i, 0)),
            pl.BlockSpec((RB, PW), lambda i: (i, 0)),
            pl.BlockSpec((RB, 16), lambda i: (i, 0)),
        ],
        out_shape=[
            jax.ShapeDtypeStruct((N, D), jnp.float32),
            jax.ShapeDtypeStruct((N, D), jnp.float32),
            jax.ShapeDtypeStruct((N, PW), jnp.float32),
            jax.ShapeDtypeStruct((N, 16), jnp.float32),
        ],
    )(x, E, W_pe.reshape(1, D), b_pe.reshape(1, D), W_gat, Axs, Aad)


def _gat_edge_body(src_hbm, dst_hbm, xwa_hbm, ad_hbm, zeros_hbm, out_hbm,
                   srcv0a, srcv0b, srcv0c, dstv0a, dstv0b, dstv0c,
                   xwav0, adv0, outv0,
                   srcv1a, srcv1b, srcv1c, dstv1a, dstv1b, dstv1c,
                   xwav1, adv1, outv1,
                   comb, gsem0, gsem1, ssem0, ssem1, isem0, isem1):
    c = lax.axis_index("c")
    s = lax.axis_index("s")
    wid = s * SC_NC + c

    pltpu.sync_copy(zeros_hbm, comb.at[pl.ds(s * ROWS_PER_TILE, ROWS_PER_TILE)])
    plsc.subcore_barrier()

    iota = lax.iota(jnp.int32, 16)
    lane2 = iota < 2
    bdn = lax.GatherDimensionNumbers(
        offset_dims=(), collapsed_slice_dims=(0,), start_index_map=(0,))

    def lane_bcast(v, k):
        return lax.gather(v, (iota * 0 + k)[:, None], bdn, (1,),
                          mode=lax.GatherScatterMode.PROMISE_IN_BOUNDS)

    slots = (((srcv0a, srcv0b, srcv0c), (dstv0a, dstv0b, dstv0c),
              xwav0, adv0, outv0, gsem0, ssem0, isem0),
             ((srcv1a, srcv1b, srcv1c), (dstv1a, dstv1b, dstv1c),
              xwav1, adv1, outv1, gsem1, ssem1, isem1))

    def fetch_idx(base, srcv, dstv, isem):
        pltpu.async_copy(src_hbm.at[pl.ds(base, CHUNK)], srcv, isem)
        pltpu.async_copy(dst_hbm.at[pl.ds(base, CHUNK)], dstv, isem)

    def wait_idx(srcv, dstv, isem):
        pltpu.make_async_copy(src_hbm.at[pl.ds(0, CHUNK)], srcv, isem).wait()
        pltpu.make_async_copy(dst_hbm.at[pl.ds(0, CHUNK)], dstv, isem).wait()

    def start_gathers(srcv, dstv, xwav, adv, gsem):
        pltpu.async_copy(xwa_hbm.at[srcv], xwav, gsem)
        pltpu.async_copy(ad_hbm.at[dstv], adv, gsem)

    def wait_gathers(xwav, adv, gsem):
        pltpu.make_async_copy(xwa_hbm.at[srcv0a], xwav, gsem).wait()
        pltpu.make_async_copy(ad_hbm.at[dstv0a], adv, gsem).wait()

    def wait_scatter(outv, ssem):
        pltpu.make_async_copy(xwa_hbm.at[srcv0a], outv, ssem).wait()

    def compute(xwav, adv, outv):
        def edge_body(e4, carry):
            for k in range(4):
                e = e4 * 4 + k
                xa = xwav[e, pl.ds(XW, 16)]
                ad16 = adv[e, pl.ds(0, 16)]
                sv = jnp.where(lane2, xa + ad16, 0.0)
                ex = jnp.exp(jnp.where(sv >= 0.0, sv, 0.2 * sv))
                exm = jnp.where(lane2, ex, 0.0)
                e0 = lane_bcast(ex, 0)
                e1 = lane_bcast(ex, 1)
                outv[e, pl.ds(0, 16)] = xwav[e, pl.ds(0, 16)] * e0
                outv[e, pl.ds(16, 16)] = xwav[e, pl.ds(16, 16)] * e0
                outv[e, pl.ds(32, 16)] = xwav[e, pl.ds(32, 16)] * e1
                outv[e, pl.ds(48, 16)] = xwav[e, pl.ds(48, 16)] * e1
                outv[e, pl.ds(XW, 16)] = exm
            return carry

        lax.fori_loop(0, CHUNK // 4, edge_body, 0)

    def do_pair(pj, par):
        cur, nxt, nn2 = par, (par + 1) % 3, (par + 2) % 3
        for b in range(2):
            srcs, dsts, xwav, adv, outv, gsem, ssem, isem = slots[b]
            wait_gathers(xwav, adv, gsem)
            wait_scatter(outv, ssem)
            compute(xwav, adv, outv)
            pltpu.async_copy(outv, comb.at[dsts[cur]], ssem, add=True)
            wait_idx(srcs[nxt], dsts[nxt], isem)
            start_gathers(srcs[nxt], dsts[nxt], xwav, adv, gsem)
            nbase = wid * EPW + (pj + 2) * (2 * CHUNK) + b * CHUNK
            nbase = jnp.minimum(nbase, NUM_EDGES - CHUNK)
            fetch_idx(nbase, srcs[nn2], dsts[nn2], isem)

    def pair3_body(q, carry):
        do_pair(3 * q, 0)
        do_pair(3 * q + 1, 1)
        do_pair(3 * q + 2, 2)
        return carry

    for b in range(2):
        srcs, dsts, xwav, adv, outv, gsem, ssem, isem = slots[b]
        pltpu.sync_copy(src_hbm.at[pl.ds(wid * EPW + b * CHUNK, CHUNK)],
                        srcs[0])
        pltpu.sync_copy(dst_hbm.at[pl.ds(wid * EPW + b * CHUNK, CHUNK)],
                        dsts[0])
        start_gathers(srcs[0], dsts[0], xwav, adv, gsem)
        fetch_idx(wid * EPW + 2 * CHUNK + b * CHUNK, srcs[1], dsts[1], isem)
        pltpu.async_copy(zeros_hbm.at[pl.ds(0, CHUNK)], outv, ssem)

    lax.fori_loop(0, FULL_CHUNKS // 6, pair3_body, 0)

    for b in range(2):
        srcs, dsts, xwav, adv, outv, gsem, ssem, isem = slots[b]
        wait_gathers(xwav, adv, gsem)
        wait_scatter(outv, ssem)
        wait_idx(srcs[0], dsts[0], isem)

    @pl.when(wid < TAIL_CHUNKS)
    def _():
        srcs, dsts, xwav, adv, outv, gsem, ssem, isem = slots[0]
        base = TAIL_BASE + wid * CHUNK
        pltpu.sync_copy(src_hbm.at[pl.ds(base, CHUNK)], srcs[0])
        pltpu.sync_copy(dst_hbm.at[pl.ds(base, CHUNK)], dsts[0])
        start_gathers(srcs[0], dsts[0], xwav, adv, gsem)
        wait_gathers(xwav, adv, gsem)
        compute(xwav, adv, outv)
        pltpu.sync_copy(outv, comb.at[dsts[0]], add=True)

    plsc.subcore_barrier()
    pltpu.sync_copy(comb.at[pl.ds(s * ROWS_PER_TILE, ROWS_PER_TILE)],
                    out_hbm.at[c, pl.ds(s * ROWS_PER_TILE, ROWS_PER_TILE)])


@functools.lru_cache(maxsize=1)
def _gat_edge_pass_fn():
    return functools.partial(
        pl.kernel,
        mesh=plsc.VectorSubcoreMesh(core_axis_name="c", subcore_axis_name="s"),
        compiler_params=pltpu.CompilerParams(use_tc_tiling_on_sc=False,
                                             needs_layout_passes=False),
        out_type=jax.ShapeDtypeStruct((SC_NC, NPAD, PW), jnp.float32),
        scratch_types=[
            pltpu.VMEM((CHUNK,), jnp.int32),
            pltpu.VMEM((CHUNK,), jnp.int32),
            pltpu.VMEM((CHUNK,), jnp.int32),
            pltpu.VMEM((CHUNK,), jnp.int32),
            pltpu.VMEM((CHUNK,), jnp.int32),
            pltpu.VMEM((CHUNK,), jnp.int32),
            pltpu.VMEM((CHUNK, PW), jnp.float32),
            pltpu.VMEM((CHUNK, 16), jnp.float32),
            pltpu.VMEM((CHUNK, PW), jnp.float32),
            pltpu.VMEM((CHUNK,), jnp.int32),
            pltpu.VMEM((CHUNK,), jnp.int32),
            pltpu.VMEM((CHUNK,), jnp.int32),
            pltpu.VMEM((CHUNK,), jnp.int32),
            pltpu.VMEM((CHUNK,), jnp.int32),
            pltpu.VMEM((CHUNK,), jnp.int32),
            pltpu.VMEM((CHUNK, PW), jnp.float32),
            pltpu.VMEM((CHUNK, 16), jnp.float32),
            pltpu.VMEM((CHUNK, PW), jnp.float32),
            pltpu.VMEM_SHARED((NPAD, PW), jnp.float32),
            pltpu.SemaphoreType.DMA,
            pltpu.SemaphoreType.DMA,
            pltpu.SemaphoreType.DMA,
            pltpu.SemaphoreType.DMA,
            pltpu.SemaphoreType.DMA,
            pltpu.SemaphoreType.DMA,
        ],
    )(_gat_edge_body)


def _gat_edge_pass(src, dst, xwa, ad, zeros_sc):
    return _gat_edge_pass_fn()(src, dst, xwa, ad, zeros_sc)


def _main_body(y_ref, xwa_ref, comb_ref, base_ref, allowed_ref, e_ref, en_ref,
               zctx_ref, w1_ref, b1_ref, w2_ref, b2_ref, wt_ref, bt_ref,
               bgat_ref, wgat_ref, axs_ref, aad_ref, tv_ref,
               ynext_ref, xwan_ref, adn_ref):
    y = y_ref[...]
    xwa = xwa_ref[...]
    cmb = comb_ref[...]
    acc = cmb[0] + cmb[1]

    asn = xwa[:, XW:XW + 2]
    adn = xwa[:, XW + 2:XW + 4]
    ssl = asn + adn
    ex_self = jnp.exp(jnp.where(ssl >= 0.0, ssl, 0.2 * ssl))

    ssum0 = acc[:, XW:XW + 1] + ex_self[:, 0:1] + 1e-16
    ssum1 = acc[:, XW + 1:XW + 2] + ex_self[:, 1:2] + 1e-16
    num0 = acc[:, 0:D] + ex_self[:, 0:1] * xwa[:, 0:D]
    num1 = acc[:, D:XW] + ex_self[:, 1:2] * xwa[:, D:XW]
    social = 0.5 * (num0 / ssum0 + num1 / ssum1) + bgat_ref[...]

    E = e_ref[...]
    En = en_ref[...]
    yE = lax.dot_general(y, E, (((1,), (1,)), ((), ())))
    d2m = jnp.sum(y * y, axis=1, keepdims=True) - 2.0 * yE + En
    m = jnp.min(d2m, axis=1, keepdims=True)
    il = lax.broadcasted_iota(jnp.int32, (RB, Z), 1)
    zi = jnp.min(jnp.where(d2m == m, il, Z), axis=1, keepdims=True)
    Pf = (il == zi).astype(jnp.float32)
    Pb = (il == zi).astype(jnp.bfloat16)

    Ezi = jnp.dot(Pf, E, preferred_element_type=jnp.float32)
    En_zi = jnp.sum(Pf * En, axis=1, keepdims=True)
    loc = jnp.dot(Pf, zctx_ref[...], preferred_element_type=jnp.float32)
    maskf = jnp.dot(Pb, allowed_ref[...], preferred_element_type=jnp.float32)

    t = tv_ref[0, 0]
    dt = tv_ref[0, 1]
    t_enc = t * wt_ref[...] + bt_ref[...]
    fi = jnp.concatenate(
        [social, loc, base_ref[...], jnp.broadcast_to(t_enc, (RB, D))], axis=1)
    h1 = jnp.maximum(
        jnp.dot(fi, w1_ref[...], preferred_element_type=jnp.float32)
        + b1_ref[...], 0.0)
    desired = jnp.dot(h1, w2_ref[...], preferred_element_type=jnp.float32) \
        + b2_ref[...]

    A = lax.dot_general(desired, E, (((1,), (1,)), ((), ())))
    bsel = jnp.sum(desired * Ezi, axis=1, keepdims=True)
    dz2 = En_zi + En - 2.0 * lax.dot_general(Ezi, E, (((1,), (1,)), ((), ())))
    okz = dz2 > 1e-12
    Dn = jnp.where(okz, jnp.sqrt(jnp.where(okz, dz2, 1.0)), 0.0)
    safe = jnp.where(Dn > 1e-6, Dn, 1.0)
    proj = (A - bsel) / safe
    is_allowed = maskf > 0.5
    lmax = jnp.max(jnp.where(is_allowed, proj, -jnp.inf), axis=1, keepdims=True)
    ez = jnp.where(is_allowed, jnp.exp(proj - lmax), 0.0)
    w = ez / jnp.sum(ez, axis=1, keepdims=True)
    u = w / safe
    su = jnp.sum(u, axis=1, keepdims=True)
    fv = jnp.dot(u, E, preferred_element_type=jnp.float32) - su * Ezi
    ynew = y + dt * (0.1 * fv)
    ynext_ref[...] = ynew

    xw = jnp.dot(ynew, wgat_ref[...], preferred_element_type=jnp.float32)
    extra = jnp.dot(xw, axs_ref[...], preferred_element_type=jnp.float32)
    xwan_ref[...] = jnp.concatenate([xw, extra], axis=1)
    adn_ref[...] = jnp.dot(xw, aad_ref[...], preferred_element_type=jnp.float32)


def _main_step(y, xwa, comb, base, allowed_bf, E, En, zctx,
               W1, b1, W2, b2, W_t, b_t, b_gat, W_gat, Axs, Aad, tv):
    return pl.pallas_call(
        _main_body,
        grid=(NBLK,),
        in_specs=[
            pl.BlockSpec((RB, D), lambda i: (i, 0)),
            pl.BlockSpec((RB, PW), lambda i: (i, 0)),
            pl.BlockSpec((2, RB, PW), lambda i: (0, i, 0)),
            pl.BlockSpec((RB, D), lambda i: (i, 0)),
            pl.BlockSpec((Z, Z), lambda i: (0, 0)),
            pl.BlockSpec((Z, D), lambda i: (0, 0)),
            pl.BlockSpec((1, Z), lambda i: (0, 0)),
            pl.BlockSpec((Z, D), lambda i: (0, 0)),
            pl.BlockSpec((FF, FF), lambda i: (0, 0)),
            pl.BlockSpec((1, FF), lambda i: (0, 0)),
            pl.BlockSpec((FF, D), lambda i: (0, 0)),
            pl.BlockSpec((1, D), lambda i: (0, 0)),
            pl.BlockSpec((1, D), lambda i: (0, 0)),
            pl.BlockSpec((1, D), lambda i: (0, 0)),
            pl.BlockSpec((1, D), lambda i: (0, 0)),
            pl.BlockSpec((D, XW), lambda i: (0, 0)),
            pl.BlockSpec((XW, 16), lambda i: (0, 0)),
            pl.BlockSpec((XW, 16), lambda i: (0, 0)),
            pl.BlockSpec(memory_space=pltpu.SMEM),
        ],
        out_specs=[
            pl.BlockSpec((RB, D), lambda i: (i, 0)),
            pl.BlockSpec((RB, PW), lambda i: (i, 0)),
            pl.BlockSpec((RB, 16), lambda i: (i, 0)),
        ],
        out_shape=[
            jax.ShapeDtypeStruct((N, D), jnp.float32),
            jax.ShapeDtypeStruct((N, PW), jnp.float32),
            jax.ShapeDtypeStruct((N, 16), jnp.float32),
        ],
    )(y, xwa, comb, base, allowed_bf, E, En, zctx,
      W1, b1.reshape(1, FF), W2, b2.reshape(1, D), W_t.reshape(1, D),
      b_t.reshape(1, D), b_gat.reshape(1, D), W_gat, Axs, Aad, tv)


def kernel(x, edge_index, loc_edge_index, times, E, W_pe, b_pe, W_gat, a_src,
           a_dst, b_gat, W_gcn, b_gcn, W_t, b_t, W1, b1, W2, b2):
    src = edge_index[0]
    dst = edge_index[1]
    lsrc = loc_edge_index[0]
    ldst = loc_edge_index[1]

    zero16 = jnp.zeros((D, 16), jnp.float32)
    Axs = jnp.concatenate([
        jnp.stack([a_src[0], jnp.zeros(D), a_dst[0], jnp.zeros(D)], axis=1),
        jnp.stack([jnp.zeros(D), a_src[1], jnp.zeros(D), a_dst[1]], axis=1),
    ], axis=0)
    Axs = jnp.concatenate([Axs, jnp.zeros((XW, 12), jnp.float32)], axis=1)
    Aad = jnp.concatenate([
        jnp.stack([a_dst[0], jnp.zeros(D)], axis=1),
        jnp.stack([jnp.zeros(D), a_dst[1]], axis=1),
    ], axis=0)
    Aad = jnp.concatenate([Aad, jnp.zeros((XW, 14), jnp.float32)], axis=1)

    allowed_bf, zctx, En = _zone_consts(lsrc, ldst, E, W_gcn, b_gcn)
    init, base, xwa, ad = _node_consts(x, E, W_pe, b_pe, W_gat, Axs, Aad)

    zeros_sc = jnp.zeros((ROWS_PER_TILE, PW), jnp.float32)

    traj = [init]
    y = init
    for k in range(T - 1):
        comb = _gat_edge_pass(src, dst, xwa, ad, zeros_sc)
        tv = jnp.stack([times[k], times[k + 1] - times[k]]).reshape(1, 2)
        y, xwa, ad = _main_step(y, xwa, comb, base, allowed_bf, E, En, zctx,
                                W1, b1, W2, b2, W_t, b_t, b_gat, W_gat,
                                Axs, Aad, tv)
        traj.append(y)
    return jnp.stack(traj, axis=0)
